# baseline (speedup 1.0000x reference)
import functools

import jax
import jax.numpy as jnp
from jax import lax
from jax.experimental import pallas as pl
from jax.experimental.pallas import tpu as pltpu
from jax.experimental.pallas import tpu_sc as plsc

N = 10000
E = 320000
B = 4
MSG = 128

_NW = 32
_PER_W = E // _NW
_C = 40
_NB = 5

_BLK = 2560
_GRID = E // _BLK

_NPAD = 10240


def _proj_body(x_ref, wj_ref, wi_ref, xa_ref, xb_ref):
    x = x_ref[...]
    xa_ref[...] = jnp.dot(x, wj_ref[...], preferred_element_type=jnp.float32)
    xb_ref[...] = jnp.dot(x, wi_ref[...], preferred_element_type=jnp.float32)


def _project(x, wj, wi):
    blk = 2000
    return pl.pallas_call(
        _proj_body,
        grid=(N // blk,),
        in_specs=[
            pl.BlockSpec((blk, 128), lambda i: (i, 0)),
            pl.BlockSpec((128, 128), lambda i: (0, 0)),
            pl.BlockSpec((128, 128), lambda i: (0, 0)),
        ],
        out_specs=[
            pl.BlockSpec((blk, 128), lambda i: (i, 0)),
            pl.BlockSpec((blk, 128), lambda i: (i, 0)),
        ],
        out_shape=[
            jax.ShapeDtypeStruct((N, 128), jnp.float32),
            jax.ShapeDtypeStruct((N, 128), jnp.float32),
        ],
    )(x, wj, wi)


_K = _PER_W // _C


_NS = 2 * _NB


def _gather_body(xa_hbm, xb_hbm, idx_hbm, g_hbm, idx, ra, rb, ob, *sems):
    wid = lax.axis_index("s") * 2 + lax.axis_index("c")
    base = wid * _PER_W
    sga = sems[0:_NB]
    sgb = sems[_NB:2 * _NB]
    swb = sems[2 * _NB:3 * _NB]
    sidx = sems[3 * _NB:3 * _NB + _NS]

    def load_idx(k, slot):
        pltpu.async_copy(idx_hbm.at[wid, k], idx.at[slot], sidx[slot])

    def wait_idx(k, slot):
        pltpu.make_async_copy(
            idx_hbm.at[wid, k], idx.at[slot], sidx[slot]).wait()

    def start_gather(slot, b):
        pltpu.async_copy(xa_hbm.at[idx.at[slot, 0]], ra.at[b], sga[b])
        pltpu.async_copy(xb_hbm.at[idx.at[slot, 1]], rb.at[b], sgb[b])

    def wait_gather(slot, b):
        pltpu.make_async_copy(
            xa_hbm.at[idx.at[slot, 0]], ra.at[b], sga[b]).wait()
        pltpu.make_async_copy(
            xb_hbm.at[idx.at[slot, 1]], rb.at[b], sgb[b]).wait()

    def vadd(b):
        def row(i, c2):
            for j in range(MSG // 16):
                s = pl.ds(j * 16, 16)
                ob[b, i, s] = ra[b, i, s] + rb[b, i, s]
            return c2
        lax.fori_loop(0, _C, row, 0)

    def start_wb(k, b):
        pltpu.async_copy(ob.at[b], g_hbm.at[pl.ds(base + k * _C, _C)], swb[b])

    def wait_wb(k, b):
        pltpu.make_async_copy(
            ob.at[b], g_hbm.at[pl.ds(base + k * _C, _C)], swb[b]).wait()

    for j in range(_NS):
        load_idx(j, j)
    for j in range(_NB):
        wait_idx(j, j)
        start_gather(j, j)

    def body(k, j, first, last):
        b = j % _NB
        wait_gather(j, b)
        if not last:
            load_idx(k + _NS, j)
        if not first:
            wait_wb(k - _NB, b)
        vadd(b)
        start_wb(k, b)
        if not last:
            wait_idx(k + _NB, (j + _NB) % _NS)
            start_gather((j + _NB) % _NS, b)

    for j in range(_NS):
        body(j, j, j < _NB, False)

    def group(g, carry):
        for j in range(_NS):
            body(g * _NS + j, j, False, False)
        return carry

    lax.fori_loop(1, _K // _NS - 1, group, 0)

    for j in range(_NS):
        k = _K - _NS + j
        b = j % _NB
        wait_gather(j, b)
        wait_wb(k - _NB, b)
        vadd(b)
        start_wb(k, b)
        if j < _NB:
            wait_idx(k + _NB, (j + _NB) % _NS)
            start_gather((j + _NB) % _NS, b)

    for j in range(_NB):
        wait_wb(_K - _NB + j, j % _NB)


def _gather_add(xa, xb, idx4):
    mesh = plsc.VectorSubcoreMesh(core_axis_name="c", subcore_axis_name="s")
    f = pl.kernel(
        _gather_body,
        mesh=mesh,
        out_type=jax.ShapeDtypeStruct((E, MSG), jnp.float32),
        scratch_types=[
            pltpu.VMEM((_NS, 2, _C), jnp.int32),
            pltpu.VMEM((_NB, _C, MSG), jnp.float32),
            pltpu.VMEM((_NB, _C, MSG), jnp.float32),
            pltpu.VMEM((_NB, _C, MSG), jnp.float32),
        ] + [pltpu.SemaphoreType.DMA] * (3 * _NB + _NS),
    )
    return f(xa, xb, idx4)


def _softplus(x):
    return jnp.log(1.0 + jnp.exp(x))


def _main_body(g_ref, rdT_ref, ea_ref, src_ref, batch_ref, f44_ref, wp_ref,
               wea_ref, b1_ref, bias_ref, w1_ref, en2_ref, en3_ref,
               p2_ref, p3_ref, d2_ref, d3_ref, out_ref):
    step = pl.program_id(0)

    @pl.when(step == 0)
    def _():
        out_ref[...] = jnp.zeros_like(out_ref)

    batch = batch_ref[...]
    srcv = src_ref[...].reshape(1, _BLK)
    off1 = jnp.sum((batch < 1).astype(jnp.int32))
    off2 = jnp.sum((batch < 2).astype(jnp.int32))
    off3 = jnp.sum((batch < 3).astype(jnp.int32))
    eg = ((srcv >= off1).astype(jnp.int32) + (srcv >= off2).astype(jnp.int32)
          + (srcv >= off3).astype(jnp.int32))

    onehotT = (lax.broadcasted_iota(jnp.int32, (8, 1), 0) == eg).astype(
        jnp.float32)
    feT = lax.dot_general(f44_ref[...], onehotT, (((0,), (0,)), ((), ())),
                          preferred_element_type=jnp.float32)

    rdT = rdT_ref[...]
    ri0 = rdT[0:1, :]
    ri1 = rdT[1:2, :]
    r0 = feT[0:1, :] * ri0 + feT[1:2, :] * ri1
    r1 = feT[2:3, :] * ri0 + feT[3:4, :] * ri1
    dd = jnp.sqrt(r0 * r0 + r1 * r1)

    b = bias_ref[...]
    onesT = jnp.ones_like(r0)
    partsT = jnp.concatenate([r0, r1, dd, rdT, onesT, jnp.zeros_like(r0)],
                             axis=0)
    pre = (g_ref[...]
           + lax.dot_general(partsT, wp_ref[...], (((0,), (0,)), ((), ())),
                             preferred_element_type=jnp.float32)
           + jnp.dot(ea_ref[...], wea_ref[...],
                     preferred_element_type=jnp.float32))
    m = _softplus(pre)

    def _dot(a, w_ref):
        return jnp.dot(a, w_ref[...], preferred_element_type=jnp.float32)

    z = _softplus(_dot(m, w1_ref) + b1_ref[0:1, :])
    hp = _softplus(_dot(z[:, 0:128], p2_ref) + b[5:6, :])
    pp = _dot(hp, p3_ref) + b[6:7, 0:4]
    hd = _softplus(_dot(z[:, 128:256], d2_ref) + b[8:9, :])
    pd = _dot(hd, d3_ref) + b[9:10, 0:16]
    h = _softplus(_dot(z[:, 256:320], en2_ref) + b[2:3, 0:64])
    en = _dot(h, en3_ref) + b[3:4, 0:2]

    zeros = jnp.zeros((en.shape[0], 128 - 22), jnp.float32)
    vals = jnp.concatenate([en, pp, pd, zeros], axis=1)
    part = lax.dot_general(onehotT, vals, (((1,), (0,)), ((), ())),
                           preferred_element_type=jnp.float32)
    out_ref[...] += part

    @pl.when(step == _GRID - 1)
    def _():
        acc = out_ref[...]
        cnt = jnp.maximum(acc[:, 1:2], 1.0)
        col = lax.broadcasted_iota(jnp.int32, (8, 128), 1)
        div = jnp.logical_and(col >= 2, col <= 21)
        out_ref[...] = jnp.where(div, acc / cnt, acc)


def _main(g, rdT, ea, src3, batch2d, f44, wp, wea, b1, bias, w1, en2, en3,
          p2, p3, d2, d3):
    full = lambda shape: pl.BlockSpec(shape, lambda i: (0,) * len(shape))
    return pl.pallas_call(
        _main_body,
        grid=(_GRID,),
        in_specs=[
            pl.BlockSpec((_BLK, 128), lambda i: (i, 0)),
            pl.BlockSpec((3, _BLK), lambda i: (0, i)),
            pl.BlockSpec((_BLK, 16), lambda i: (i, 0)),
            pl.BlockSpec((1, 1, _BLK), lambda i: (i, 0, 0)),
            full((80, 128)),
            full((8, 4)),
            full((8, 128)),
            full((16, 128)),
            full((8, 320)),
            full((16, 128)),
            full((128, 320)),
            full((64, 64)),
            full((64, 2)),
            full((128, 128)),
            full((128, 4)),
            full((128, 128)),
            full((128, 16)),
        ],
        out_specs=pl.BlockSpec((8, 128), lambda i: (0, 0)),
        out_shape=jax.ShapeDtypeStruct((8, 128), jnp.float32),
    )(g, rdT, ea, src3, batch2d, f44, wp, wea, b1, bias, w1, en2, en3,
      p2, p3, d2, d3)


def kernel(x, edge_attr, F, pos, r, d, mean_pos, params, edge_index, batch):
    p = params
    msg_W = p['msg_W']
    wj = msg_W[0:128]
    wi = msg_W[128:256]
    wp = jnp.zeros((8, 128), jnp.float32)
    wp = wp.at[0:6].set(msg_W[256:262])
    wp = wp.at[6].set(p['msg_b'])
    wea = msg_W[262:278]

    src = edge_index[0].astype(jnp.int32)
    dst = edge_index[1].astype(jnp.int32)
    rdT = jnp.concatenate([r.T, d.T], axis=0)
    src3 = src.reshape(_GRID, 1, _BLK)
    idx4 = jnp.stack([src.reshape(_NW, _K, _C), dst.reshape(_NW, _K, _C)],
                     axis=2)

    batch2d = jnp.concatenate(
        [batch.astype(jnp.int32), jnp.full((_NPAD - N,), B, jnp.int32)]
    ).reshape(80, 128)

    f44 = jnp.zeros((8, 4), jnp.float32).at[0:4].set(F.reshape(4, 4))

    bias = jnp.zeros((16, 128), jnp.float32)
    bias = bias.at[0, :].set(p['msg_b'])
    bias = bias.at[1, 0:64].set(p['en1_b'])
    bias = bias.at[2, 0:64].set(p['en2_b'])
    bias = bias.at[3, 0:1].set(p['en3_b'])
    bias = bias.at[3, 1].set(1.0)
    bias = bias.at[4, :].set(p['P1_b'])
    bias = bias.at[5, :].set(p['P2_b'])
    bias = bias.at[6, 0:4].set(p['P3_b'])
    bias = bias.at[7, :].set(p['D1_b'])
    bias = bias.at[8, :].set(p['D2_b'])
    bias = bias.at[9, 0:16].set(p['D3_b'])

    en3p = jnp.zeros((64, 2), jnp.float32).at[:, 0:1].set(p['en3_W'])
    w1 = jnp.concatenate([p['P1_W'], p['D1_W'], p['en1_W']], axis=1)
    b1 = jnp.zeros((8, 320), jnp.float32)
    b1 = b1.at[0, 0:128].set(p['P1_b'])
    b1 = b1.at[0, 128:256].set(p['D1_b'])
    b1 = b1.at[0, 256:320].set(p['en1_b'])

    xa, xb = _project(x, wj, wi)
    g = _gather_add(xa, xb, idx4)
    res = _main(g, rdT, edge_attr, src3, batch2d, f44, wp, wea, b1, bias,
                w1, p['en2_W'], en3p,
                p['P2_W'], p['P3_W'], p['D2_W'], p['D3_W'])
    return jnp.concatenate([res[0:4, 0:1], res[0:4, 2:22]], axis=1)


# speedup vs baseline: 1.0295x; 1.0295x over previous
import functools

import jax
import jax.numpy as jnp
from jax import lax
from jax.experimental import pallas as pl
from jax.experimental.pallas import tpu as pltpu
from jax.experimental.pallas import tpu_sc as plsc

N = 10000
E = 320000
B = 4
MSG = 128

_NW = 32
_PER_W = E // _NW
_C = 40
_NB = 5

_BLK = 2560
_GRID = E // _BLK

_NPAD = 10240


def _proj_body(x_ref, wj_ref, wi_ref, xa_ref, xb_ref):
    x = x_ref[...]
    xa_ref[...] = jnp.dot(x, wj_ref[...], preferred_element_type=jnp.float32)
    xb_ref[...] = jnp.dot(x, wi_ref[...], preferred_element_type=jnp.float32)


def _project(x, wj, wi):
    blk = 2000
    return pl.pallas_call(
        _proj_body,
        grid=(N // blk,),
        in_specs=[
            pl.BlockSpec((blk, 128), lambda i: (i, 0)),
            pl.BlockSpec((128, 128), lambda i: (0, 0)),
            pl.BlockSpec((128, 128), lambda i: (0, 0)),
        ],
        out_specs=[
            pl.BlockSpec((blk, 128), lambda i: (i, 0)),
            pl.BlockSpec((blk, 128), lambda i: (i, 0)),
        ],
        out_shape=[
            jax.ShapeDtypeStruct((N, 128), jnp.float32),
            jax.ShapeDtypeStruct((N, 128), jnp.float32),
        ],
    )(x, wj, wi)


_K = _PER_W // _C


_NS = 2 * _NB


def _gather_body(xa_hbm, xb_hbm, idx_hbm, g_hbm, idx, ra, rb, ob, *sems):
    wid = lax.axis_index("s") * 2 + lax.axis_index("c")
    base = wid * _PER_W
    sga = sems[0:_NB]
    sgb = sems[_NB:2 * _NB]
    swb = sems[2 * _NB:3 * _NB]
    sidx = sems[3 * _NB:3 * _NB + _NS]

    def load_idx(k, slot):
        pltpu.async_copy(idx_hbm.at[wid, k], idx.at[slot], sidx[slot])

    def wait_idx(k, slot):
        pltpu.make_async_copy(
            idx_hbm.at[wid, k], idx.at[slot], sidx[slot]).wait()

    def start_gather(slot, b):
        pltpu.async_copy(xa_hbm.at[idx.at[slot, 0]], ra.at[b], sga[b])
        pltpu.async_copy(xb_hbm.at[idx.at[slot, 1]], rb.at[b], sgb[b])

    def wait_gather(slot, b):
        pltpu.make_async_copy(
            xa_hbm.at[idx.at[slot, 0]], ra.at[b], sga[b]).wait()
        pltpu.make_async_copy(
            xb_hbm.at[idx.at[slot, 1]], rb.at[b], sgb[b]).wait()

    def vadd(b):
        def row(i, c2):
            for j in range(MSG // 16):
                s = pl.ds(j * 16, 16)
                ob[b, i, s] = ra[b, i, s] + rb[b, i, s]
            return c2
        lax.fori_loop(0, _C, row, 0)

    def start_wb(k, b):
        pltpu.async_copy(ob.at[b], g_hbm.at[pl.ds(base + k * _C, _C)], swb[b])

    def wait_wb(k, b):
        pltpu.make_async_copy(
            ob.at[b], g_hbm.at[pl.ds(base + k * _C, _C)], swb[b]).wait()

    for j in range(_NS):
        load_idx(j, j)
    for j in range(_NB):
        wait_idx(j, j)
        start_gather(j, j)

    def body(k, j, first, last):
        b = j % _NB
        wait_gather(j, b)
        if not last:
            load_idx(k + _NS, j)
        if not first:
            wait_wb(k - _NB, b)
        vadd(b)
        start_wb(k, b)
        if not last:
            wait_idx(k + _NB, (j + _NB) % _NS)
            start_gather((j + _NB) % _NS, b)

    for j in range(_NS):
        body(j, j, j < _NB, False)

    def group(g, carry):
        for j in range(_NS):
            body(g * _NS + j, j, False, False)
        return carry

    lax.fori_loop(1, _K // _NS - 1, group, 0)

    for j in range(_NS):
        k = _K - _NS + j
        b = j % _NB
        wait_gather(j, b)
        wait_wb(k - _NB, b)
        vadd(b)
        start_wb(k, b)
        if j < _NB:
            wait_idx(k + _NB, (j + _NB) % _NS)
            start_gather((j + _NB) % _NS, b)

    for j in range(_NB):
        wait_wb(_K - _NB + j, j % _NB)


def _gather_add(xa, xb, idx4):
    mesh = plsc.VectorSubcoreMesh(core_axis_name="c", subcore_axis_name="s")
    f = pl.kernel(
        _gather_body,
        mesh=mesh,
        out_type=jax.ShapeDtypeStruct((E, MSG), jnp.float32),
        scratch_types=[
            pltpu.VMEM((_NS, 2, _C), jnp.int32),
            pltpu.VMEM((_NB, _C, MSG), jnp.float32),
            pltpu.VMEM((_NB, _C, MSG), jnp.float32),
            pltpu.VMEM((_NB, _C, MSG), jnp.float32),
        ] + [pltpu.SemaphoreType.DMA] * (3 * _NB + _NS),
    )
    return f(xa, xb, idx4)


def _softplus(x):
    return jnp.log(1.0 + jnp.exp(x))


def _main_body(g_ref, rdT_ref, ea_ref, src_ref, batch_ref, f44_ref, wp_ref,
               wea_ref, bias_ref, en1_ref, en2_ref, en3_ref,
               p1_ref, p2_ref, p3_ref, d1_ref, d2_ref, d3_ref, out_ref):
    step = pl.program_id(0)

    @pl.when(step == 0)
    def _():
        out_ref[...] = jnp.zeros_like(out_ref)

    batch = batch_ref[...]
    srcv = src_ref[...].reshape(1, _BLK)
    off1 = jnp.sum((batch < 1).astype(jnp.int32))
    off2 = jnp.sum((batch < 2).astype(jnp.int32))
    off3 = jnp.sum((batch < 3).astype(jnp.int32))
    eg = ((srcv >= off1).astype(jnp.int32) + (srcv >= off2).astype(jnp.int32)
          + (srcv >= off3).astype(jnp.int32))

    onehotT = (lax.broadcasted_iota(jnp.int32, (8, 1), 0) == eg).astype(
        jnp.float32)
    feT = lax.dot_general(f44_ref[...], onehotT, (((0,), (0,)), ((), ())),
                          preferred_element_type=jnp.float32)

    rdT = rdT_ref[...]
    ri0 = rdT[0:1, :]
    ri1 = rdT[1:2, :]
    r0 = feT[0:1, :] * ri0 + feT[1:2, :] * ri1
    r1 = feT[2:3, :] * ri0 + feT[3:4, :] * ri1
    dd = jnp.sqrt(r0 * r0 + r1 * r1)

    b = bias_ref[...]
    onesT = jnp.ones_like(r0)
    partsT = jnp.concatenate([r0, r1, dd, rdT, onesT, jnp.zeros_like(r0)],
                             axis=0)
    pre = (g_ref[...]
           + lax.dot_general(partsT, wp_ref[...], (((0,), (0,)), ((), ())),
                             preferred_element_type=jnp.float32)
           + jnp.dot(ea_ref[...], wea_ref[...],
                     preferred_element_type=jnp.float32))
    m = _softplus(pre)

    def _dot(a, w_ref):
        return jnp.dot(a, w_ref[...], preferred_element_type=jnp.float32)

    hp = _softplus(_dot(m, p1_ref) + b[4:5, :])
    hp = _softplus(_dot(hp, p2_ref) + b[5:6, :])
    pp = _dot(hp, p3_ref) + b[6:7, 0:4]
    hd = _softplus(_dot(m, d1_ref) + b[7:8, :])
    hd = _softplus(_dot(hd, d2_ref) + b[8:9, :])
    pd = _dot(hd, d3_ref) + b[9:10, 0:16]
    h = _softplus(_dot(m, en1_ref) + b[1:2, 0:64])
    h = _softplus(_dot(h, en2_ref) + b[2:3, 0:64])
    en = _dot(h, en3_ref) + b[3:4, 0:2]

    zeros = jnp.zeros((en.shape[0], 128 - 22), jnp.float32)
    vals = jnp.concatenate([en, pp, pd, zeros], axis=1)
    part = lax.dot_general(onehotT, vals, (((1,), (0,)), ((), ())),
                           preferred_element_type=jnp.float32)
    out_ref[...] += part

    @pl.when(step == _GRID - 1)
    def _():
        acc = out_ref[...]
        cnt = jnp.maximum(acc[:, 1:2], 1.0)
        col = lax.broadcasted_iota(jnp.int32, (8, 128), 1)
        div = jnp.logical_and(col >= 2, col <= 21)
        out_ref[...] = jnp.where(div, acc / cnt, acc)


def _main(g, rdT, ea, src3, batch2d, f44, wp, wea, bias, en1, en2, en3,
          p1, p2, p3, d1, d2, d3):
    full = lambda shape: pl.BlockSpec(shape, lambda i: (0,) * len(shape))
    return pl.pallas_call(
        _main_body,
        grid=(_GRID,),
        in_specs=[
            pl.BlockSpec((_BLK, 128), lambda i: (i, 0)),
            pl.BlockSpec((3, _BLK), lambda i: (0, i)),
            pl.BlockSpec((_BLK, 16), lambda i: (i, 0)),
            pl.BlockSpec((1, 1, _BLK), lambda i: (i, 0, 0)),
            full((80, 128)),
            full((8, 4)),
            full((8, 128)),
            full((16, 128)),
            full((16, 128)),
            full((128, 64)),
            full((64, 64)),
            full((64, 2)),
            full((128, 128)),
            full((128, 128)),
            full((128, 4)),
            full((128, 128)),
            full((128, 128)),
            full((128, 16)),
        ],
        out_specs=pl.BlockSpec((8, 128), lambda i: (0, 0)),
        out_shape=jax.ShapeDtypeStruct((8, 128), jnp.float32),
    )(g, rdT, ea, src3, batch2d, f44, wp, wea, bias, en1, en2, en3,
      p1, p2, p3, d1, d2, d3)


def kernel(x, edge_attr, F, pos, r, d, mean_pos, params, edge_index, batch):
    p = params
    msg_W = p['msg_W']
    wj = msg_W[0:128]
    wi = msg_W[128:256]
    wp = jnp.zeros((8, 128), jnp.float32)
    wp = wp.at[0:6].set(msg_W[256:262])
    wp = wp.at[6].set(p['msg_b'])
    wea = msg_W[262:278]

    src = edge_index[0].astype(jnp.int32)
    dst = edge_index[1].astype(jnp.int32)
    rdT = jnp.concatenate([r.T, d.T], axis=0)
    src3 = src.reshape(_GRID, 1, _BLK)
    idx4 = jnp.stack([src.reshape(_NW, _K, _C), dst.reshape(_NW, _K, _C)],
                     axis=2)

    batch2d = jnp.concatenate(
        [batch.astype(jnp.int32), jnp.full((_NPAD - N,), B, jnp.int32)]
    ).reshape(80, 128)

    f44 = jnp.zeros((8, 4), jnp.float32).at[0:4].set(F.reshape(4, 4))

    bias = jnp.zeros((16, 128), jnp.float32)
    bias = bias.at[0, :].set(p['msg_b'])
    bias = bias.at[1, 0:64].set(p['en1_b'])
    bias = bias.at[2, 0:64].set(p['en2_b'])
    bias = bias.at[3, 0:1].set(p['en3_b'])
    bias = bias.at[3, 1].set(1.0)
    bias = bias.at[4, :].set(p['P1_b'])
    bias = bias.at[5, :].set(p['P2_b'])
    bias = bias.at[6, 0:4].set(p['P3_b'])
    bias = bias.at[7, :].set(p['D1_b'])
    bias = bias.at[8, :].set(p['D2_b'])
    bias = bias.at[9, 0:16].set(p['D3_b'])

    en3p = jnp.zeros((64, 2), jnp.float32).at[:, 0:1].set(p['en3_W'])

    xa, xb = _project(x, wj, wi)
    g = _gather_add(xa, xb, idx4)
    res = _main(g, rdT, edge_attr, src3, batch2d, f44, wp, wea, bias,
                p['en1_W'], p['en2_W'], en3p,
                p['P1_W'], p['P2_W'], p['P3_W'],
                p['D1_W'], p['D2_W'], p['D3_W'])
    return jnp.concatenate([res[0:4, 0:1], res[0:4, 2:22]], axis=1)


# speedup vs baseline: 1.1317x; 1.0992x over previous
import functools

import jax
import jax.numpy as jnp
from jax import lax
from jax.experimental import pallas as pl
from jax.experimental.pallas import tpu as pltpu
from jax.experimental.pallas import tpu_sc as plsc

N = 10000
E = 320000
B = 4
MSG = 128

_NW = 32
_PER_W = E // _NW
_C = 40
_NB = 5

_BLK = 2560
_GRID = E // _BLK

_NPAD = 10240


def _proj_body(x_ref, wj_ref, wi_ref, xa_ref, xb_ref):
    x = x_ref[...]
    xa_ref[...] = jnp.dot(x, wj_ref[...], preferred_element_type=jnp.float32)
    xb_ref[...] = jnp.dot(x, wi_ref[...], preferred_element_type=jnp.float32)


def _project(x, wj, wi):
    blk = 2000
    return pl.pallas_call(
        _proj_body,
        grid=(N // blk,),
        in_specs=[
            pl.BlockSpec((blk, 128), lambda i: (i, 0)),
            pl.BlockSpec((128, 128), lambda i: (0, 0)),
            pl.BlockSpec((128, 128), lambda i: (0, 0)),
        ],
        out_specs=[
            pl.BlockSpec((blk, 128), lambda i: (i, 0)),
            pl.BlockSpec((blk, 128), lambda i: (i, 0)),
        ],
        out_shape=[
            jax.ShapeDtypeStruct((N, 128), jnp.float32),
            jax.ShapeDtypeStruct((N, 128), jnp.float32),
        ],
    )(x, wj, wi)


_K = _PER_W // _C


_NS = 2 * _NB


def _gather_body(xa_hbm, xb_hbm, idx_hbm, g_hbm, idx, ra, rb, ob, *sems):
    wid = lax.axis_index("s") * 2 + lax.axis_index("c")
    base = wid * _PER_W
    sga = sems[0:_NB]
    sgb = sems[_NB:2 * _NB]
    swb = sems[2 * _NB:3 * _NB]
    sidx = sems[3 * _NB:3 * _NB + _NS]

    def load_idx(k, slot):
        pltpu.async_copy(idx_hbm.at[wid, k], idx.at[slot], sidx[slot])

    def wait_idx(k, slot):
        pltpu.make_async_copy(
            idx_hbm.at[wid, k], idx.at[slot], sidx[slot]).wait()

    def start_gather(slot, b):
        pltpu.async_copy(xa_hbm.at[idx.at[slot, 0]], ra.at[b], sga[b])
        pltpu.async_copy(xb_hbm.at[idx.at[slot, 1]], rb.at[b], sgb[b])

    def wait_gather(slot, b):
        pltpu.make_async_copy(
            xa_hbm.at[idx.at[slot, 0]], ra.at[b], sga[b]).wait()
        pltpu.make_async_copy(
            xb_hbm.at[idx.at[slot, 1]], rb.at[b], sgb[b]).wait()

    def vadd(b):
        def row(i, c2):
            for j in range(MSG // 16):
                s = pl.ds(j * 16, 16)
                ob[b, i, s] = ra[b, i, s] + rb[b, i, s]
            return c2
        lax.fori_loop(0, _C, row, 0)

    def start_wb(k, b):
        pltpu.async_copy(ob.at[b], g_hbm.at[pl.ds(base + k * _C, _C)], swb[b])

    def wait_wb(k, b):
        pltpu.make_async_copy(
            ob.at[b], g_hbm.at[pl.ds(base + k * _C, _C)], swb[b]).wait()

    for j in range(_NS):
        load_idx(j, j)
    for j in range(_NB):
        wait_idx(j, j)
        start_gather(j, j)

    def body(k, j, first, last):
        b = j % _NB
        wait_gather(j, b)
        if not last:
            load_idx(k + _NS, j)
        if not first:
            wait_wb(k - _NB, b)
        vadd(b)
        start_wb(k, b)
        if not last:
            wait_idx(k + _NB, (j + _NB) % _NS)
            start_gather((j + _NB) % _NS, b)

    for j in range(_NS):
        body(j, j, j < _NB, False)

    def group(g, carry):
        for j in range(_NS):
            body(g * _NS + j, j, False, False)
        return carry

    lax.fori_loop(1, _K // _NS - 1, group, 0)

    for j in range(_NS):
        k = _K - _NS + j
        b = j % _NB
        wait_gather(j, b)
        wait_wb(k - _NB, b)
        vadd(b)
        start_wb(k, b)
        if j < _NB:
            wait_idx(k + _NB, (j + _NB) % _NS)
            start_gather((j + _NB) % _NS, b)

    for j in range(_NB):
        wait_wb(_K - _NB + j, j % _NB)


def _gather_add(xa, xb, idx4):
    mesh = plsc.VectorSubcoreMesh(core_axis_name="c", subcore_axis_name="s")
    f = pl.kernel(
        _gather_body,
        mesh=mesh,
        out_type=jax.ShapeDtypeStruct((E, MSG), jnp.float32),
        scratch_types=[
            pltpu.VMEM((_NS, 2, _C), jnp.int32),
            pltpu.VMEM((_NB, _C, MSG), jnp.float32),
            pltpu.VMEM((_NB, _C, MSG), jnp.float32),
            pltpu.VMEM((_NB, _C, MSG), jnp.float32),
        ] + [pltpu.SemaphoreType.DMA] * (3 * _NB + _NS),
    )
    return f(xa, xb, idx4)


def _softplus(x):
    return jnp.log(1.0 + jnp.exp(x))


def _main_body(g_ref, featT_ref, src_ref, batch_ref, f44_ref, wp_ref,
               bias_ref, en1_ref, en2_ref, en3_ref,
               p1_ref, p2_ref, p3_ref, d1_ref, d2_ref, d3_ref, out_ref):
    step = pl.program_id(0)

    @pl.when(step == 0)
    def _():
        out_ref[...] = jnp.zeros_like(out_ref)

    batch = batch_ref[...]
    srcv = src_ref[...].reshape(1, _BLK)
    off1 = jnp.sum((batch < 1).astype(jnp.int32))
    off2 = jnp.sum((batch < 2).astype(jnp.int32))
    off3 = jnp.sum((batch < 3).astype(jnp.int32))
    eg = ((srcv >= off1).astype(jnp.int32) + (srcv >= off2).astype(jnp.int32)
          + (srcv >= off3).astype(jnp.int32))

    onehotT = (lax.broadcasted_iota(jnp.int32, (8, 1), 0) == eg).astype(
        jnp.float32)
    feT = lax.dot_general(f44_ref[...], onehotT, (((0,), (0,)), ((), ())),
                          preferred_element_type=jnp.float32)

    featT = featT_ref[...]
    ri0 = featT[0:1, :]
    ri1 = featT[1:2, :]
    r0 = feT[0:1, :] * ri0 + feT[1:2, :] * ri1
    r1 = feT[2:3, :] * ri0 + feT[3:4, :] * ri1
    dd = jnp.sqrt(r0 * r0 + r1 * r1)

    b = bias_ref[...]
    onesT = jnp.ones_like(r0)
    partsT = jnp.concatenate([r0, r1, dd, featT, onesT, jnp.zeros_like(r0)],
                             axis=0)
    pre = g_ref[...] + lax.dot_general(
        partsT, wp_ref[...], (((0,), (0,)), ((), ())),
        preferred_element_type=jnp.float32)
    m = _softplus(pre)

    def _dot(a, w_ref):
        return jnp.dot(a, w_ref[...], preferred_element_type=jnp.float32)

    hp = _softplus(_dot(m, p1_ref) + b[4:5, :])
    hp = _softplus(_dot(hp, p2_ref) + b[5:6, :])
    pp = _dot(hp, p3_ref) + b[6:7, 0:4]
    hd = _softplus(_dot(m, d1_ref) + b[7:8, :])
    hd = _softplus(_dot(hd, d2_ref) + b[8:9, :])
    pd = _dot(hd, d3_ref) + b[9:10, 0:16]
    h = _softplus(_dot(m, en1_ref) + b[1:2, 0:64])
    h = _softplus(_dot(h, en2_ref) + b[2:3, 0:64])
    en = _dot(h, en3_ref) + b[3:4, 0:2]

    zeros = jnp.zeros((en.shape[0], 128 - 22), jnp.float32)
    vals = jnp.concatenate([en, pp, pd, zeros], axis=1)
    part = lax.dot_general(onehotT, vals, (((1,), (0,)), ((), ())),
                           preferred_element_type=jnp.float32)
    out_ref[...] += part

    @pl.when(step == _GRID - 1)
    def _():
        acc = out_ref[...]
        cnt = jnp.maximum(acc[:, 1:2], 1.0)
        col = lax.broadcasted_iota(jnp.int32, (8, 128), 1)
        div = jnp.logical_and(col >= 2, col <= 21)
        out_ref[...] = jnp.where(div, acc / cnt, acc)


def _main(g, featT, src3, batch2d, f44, wp, bias, en1, en2, en3,
          p1, p2, p3, d1, d2, d3):
    full = lambda shape: pl.BlockSpec(shape, lambda i: (0,) * len(shape))
    return pl.pallas_call(
        _main_body,
        grid=(_GRID,),
        in_specs=[
            pl.BlockSpec((_BLK, 128), lambda i: (i, 0)),
            pl.BlockSpec((19, _BLK), lambda i: (0, i)),
            pl.BlockSpec((1, 1, _BLK), lambda i: (i, 0, 0)),
            full((80, 128)),
            full((8, 4)),
            full((24, 128)),
            full((16, 128)),
            full((128, 64)),
            full((64, 64)),
            full((64, 2)),
            full((128, 128)),
            full((128, 128)),
            full((128, 4)),
            full((128, 128)),
            full((128, 128)),
            full((128, 16)),
        ],
        out_specs=pl.BlockSpec((8, 128), lambda i: (0, 0)),
        out_shape=jax.ShapeDtypeStruct((8, 128), jnp.float32),
    )(g, featT, src3, batch2d, f44, wp, bias, en1, en2, en3,
      p1, p2, p3, d1, d2, d3)


def kernel(x, edge_attr, F, pos, r, d, mean_pos, params, edge_index, batch):
    p = params
    msg_W = p['msg_W']
    wj = msg_W[0:128]
    wi = msg_W[128:256]
    wp = jnp.zeros((24, 128), jnp.float32)
    wp = wp.at[0:22].set(msg_W[256:278])
    wp = wp.at[22].set(p['msg_b'])

    src = edge_index[0].astype(jnp.int32)
    dst = edge_index[1].astype(jnp.int32)
    featT = jnp.concatenate([r.T, d.T, edge_attr.T], axis=0)
    src3 = src.reshape(_GRID, 1, _BLK)
    idx4 = jnp.stack([src.reshape(_NW, _K, _C), dst.reshape(_NW, _K, _C)],
                     axis=2)

    batch2d = jnp.concatenate(
        [batch.astype(jnp.int32), jnp.full((_NPAD - N,), B, jnp.int32)]
    ).reshape(80, 128)

    f44 = jnp.zeros((8, 4), jnp.float32).at[0:4].set(F.reshape(4, 4))

    bias = jnp.zeros((16, 128), jnp.float32)
    bias = bias.at[0, :].set(p['msg_b'])
    bias = bias.at[1, 0:64].set(p['en1_b'])
    bias = bias.at[2, 0:64].set(p['en2_b'])
    bias = bias.at[3, 0:1].set(p['en3_b'])
    bias = bias.at[3, 1].set(1.0)
    bias = bias.at[4, :].set(p['P1_b'])
    bias = bias.at[5, :].set(p['P2_b'])
    bias = bias.at[6, 0:4].set(p['P3_b'])
    bias = bias.at[7, :].set(p['D1_b'])
    bias = bias.at[8, :].set(p['D2_b'])
    bias = bias.at[9, 0:16].set(p['D3_b'])

    en3p = jnp.zeros((64, 2), jnp.float32).at[:, 0:1].set(p['en3_W'])

    xa, xb = _project(x, wj, wi)
    g = _gather_add(xa, xb, idx4)
    res = _main(g, featT, src3, batch2d, f44, wp, bias,
                p['en1_W'], p['en2_W'], en3p,
                p['P1_W'], p['P2_W'], p['P3_W'],
                p['D1_W'], p['D2_W'], p['D3_W'])
    return jnp.concatenate([res[0:4, 0:1], res[0:4, 2:22]], axis=1)


# speedup vs baseline: 1.1438x; 1.0108x over previous
import functools

import jax
import jax.numpy as jnp
from jax import lax
from jax.experimental import pallas as pl
from jax.experimental.pallas import tpu as pltpu
from jax.experimental.pallas import tpu_sc as plsc

N = 10000
E = 320000
B = 4
MSG = 128

_NW = 32
_PER_W = E // _NW
_C = 40
_NB = 5

_BLK = 2560
_GRID = E // _BLK

_NPAD = 10240


def _proj_body(x_ref, wj_ref, wi_ref, xa_ref, xb_ref):
    x = x_ref[...]
    xa_ref[...] = jnp.dot(x, wj_ref[...], preferred_element_type=jnp.float32)
    xb_ref[...] = jnp.dot(x, wi_ref[...], preferred_element_type=jnp.float32)


def _project(x, wj, wi):
    blk = 2000
    return pl.pallas_call(
        _proj_body,
        grid=(N // blk,),
        in_specs=[
            pl.BlockSpec((blk, 128), lambda i: (i, 0)),
            pl.BlockSpec((128, 128), lambda i: (0, 0)),
            pl.BlockSpec((128, 128), lambda i: (0, 0)),
        ],
        out_specs=[
            pl.BlockSpec((blk, 128), lambda i: (i, 0)),
            pl.BlockSpec((blk, 128), lambda i: (i, 0)),
        ],
        out_shape=[
            jax.ShapeDtypeStruct((N, 128), jnp.float32),
            jax.ShapeDtypeStruct((N, 128), jnp.float32),
        ],
    )(x, wj, wi)


_K = _PER_W // _C


_NS = 2 * _NB


def _gather_body(xa_hbm, xb_hbm, idx_hbm, g_hbm, idx, ra, rb, ob, *sems):
    wid = lax.axis_index("s") * 2 + lax.axis_index("c")
    base = wid * _PER_W
    sga = sems[0:_NB]
    sgb = sems[_NB:2 * _NB]
    swb = sems[2 * _NB:3 * _NB]
    sidx = sems[3 * _NB:3 * _NB + _NS]

    def load_idx(k, slot):
        pltpu.async_copy(idx_hbm.at[wid, k], idx.at[slot], sidx[slot])

    def wait_idx(k, slot):
        pltpu.make_async_copy(
            idx_hbm.at[wid, k], idx.at[slot], sidx[slot]).wait()

    def start_gather(slot, b):
        pltpu.async_copy(xa_hbm.at[idx.at[slot, 0]], ra.at[b], sga[b])
        pltpu.async_copy(xb_hbm.at[idx.at[slot, 1]], rb.at[b], sgb[b])

    def wait_gather(slot, b):
        pltpu.make_async_copy(
            xa_hbm.at[idx.at[slot, 0]], ra.at[b], sga[b]).wait()
        pltpu.make_async_copy(
            xb_hbm.at[idx.at[slot, 1]], rb.at[b], sgb[b]).wait()

    def vadd(b):
        def row(i, c2):
            for j in range(MSG // 16):
                s = pl.ds(j * 16, 16)
                ob[b, i, s] = ra[b, i, s] + rb[b, i, s]
            return c2
        lax.fori_loop(0, _C, row, 0)

    def start_wb(k, b):
        pltpu.async_copy(ob.at[b], g_hbm.at[pl.ds(base + k * _C, _C)], swb[b])

    def wait_wb(k, b):
        pltpu.make_async_copy(
            ob.at[b], g_hbm.at[pl.ds(base + k * _C, _C)], swb[b]).wait()

    for j in range(_NS):
        load_idx(j, j)
    for j in range(_NB):
        wait_idx(j, j)
        start_gather(j, j)

    def body(k, j, first, last):
        b = j % _NB
        wait_gather(j, b)
        if not last:
            load_idx(k + _NS, j)
        if not first:
            wait_wb(k - _NB, b)
        vadd(b)
        start_wb(k, b)
        if not last:
            wait_idx(k + _NB, (j + _NB) % _NS)
            start_gather((j + _NB) % _NS, b)

    for j in range(_NS):
        body(j, j, j < _NB, False)

    def group(g, carry):
        for j in range(_NS):
            body(g * _NS + j, j, False, False)
        return carry

    lax.fori_loop(1, _K // _NS - 1, group, 0)

    for j in range(_NS):
        k = _K - _NS + j
        b = j % _NB
        wait_gather(j, b)
        wait_wb(k - _NB, b)
        vadd(b)
        start_wb(k, b)
        if j < _NB:
            wait_idx(k + _NB, (j + _NB) % _NS)
            start_gather((j + _NB) % _NS, b)

    for j in range(_NB):
        wait_wb(_K - _NB + j, j % _NB)


def _gather_add(xa, xb, idx4):
    mesh = plsc.VectorSubcoreMesh(core_axis_name="c", subcore_axis_name="s")
    f = pl.kernel(
        _gather_body,
        mesh=mesh,
        out_type=jax.ShapeDtypeStruct((E, MSG), jnp.float32),
        scratch_types=[
            pltpu.VMEM((_NS, 2, _C), jnp.int32),
            pltpu.VMEM((_NB, _C, MSG), jnp.float32),
            pltpu.VMEM((_NB, _C, MSG), jnp.float32),
            pltpu.VMEM((_NB, _C, MSG), jnp.float32),
        ] + [pltpu.SemaphoreType.DMA] * (3 * _NB + _NS),
    )
    return f(xa, xb, idx4)


def _softplus(x):
    return jnp.log(1.0 + jnp.exp(x))


def _main_body(g_ref, featT_ref, src_ref, batch_ref, f44_ref, wp_ref,
               bias_ref, en1_ref, en2_ref, en3_ref,
               p1_ref, p2_ref, p3_ref, d1_ref, d2_ref, d3_ref, out_ref):
    step = pl.program_id(0)

    @pl.when(step == 0)
    def _():
        out_ref[...] = jnp.zeros_like(out_ref)

    batch = batch_ref[...]
    srcv = src_ref[...].reshape(1, _BLK)
    off1 = jnp.sum((batch < 1).astype(jnp.int32))
    off2 = jnp.sum((batch < 2).astype(jnp.int32))
    off3 = jnp.sum((batch < 3).astype(jnp.int32))
    eg = ((srcv >= off1).astype(jnp.int32) + (srcv >= off2).astype(jnp.int32)
          + (srcv >= off3).astype(jnp.int32))

    onehotT = (lax.broadcasted_iota(jnp.int32, (8, 1), 0) == eg).astype(
        jnp.float32)
    feT = lax.dot_general(f44_ref[...], onehotT, (((0,), (0,)), ((), ())),
                          preferred_element_type=jnp.float32)

    featT = featT_ref[...]
    ri0 = featT[0:1, :]
    ri1 = featT[1:2, :]
    r0 = feT[0:1, :] * ri0 + feT[1:2, :] * ri1
    r1 = feT[2:3, :] * ri0 + feT[3:4, :] * ri1
    dd = jnp.sqrt(r0 * r0 + r1 * r1)

    b = bias_ref[...]
    onesT = jnp.ones_like(r0)
    partsT = jnp.concatenate([r0, r1, dd, featT, onesT, jnp.zeros_like(r0)],
                             axis=0)
    pre = g_ref[...] + lax.dot_general(
        partsT, wp_ref[...], (((0,), (0,)), ((), ())),
        preferred_element_type=jnp.float32)

    def _sp_bf(x):
        xb = x.astype(jnp.bfloat16)
        return jnp.log(1.0 + jnp.exp(xb))

    m = _sp_bf(pre)

    def _dot(a, w_ref):
        return jnp.dot(a, w_ref[...], preferred_element_type=jnp.float32)

    hp = _sp_bf(_dot(m, p1_ref) + b[4:5, :])
    hp = _sp_bf(_dot(hp, p2_ref) + b[5:6, :])
    pp = _dot(hp, p3_ref) + b[6:7, 0:4]
    hd = _sp_bf(_dot(m, d1_ref) + b[7:8, :])
    hd = _sp_bf(_dot(hd, d2_ref) + b[8:9, :])
    pd = _dot(hd, d3_ref) + b[9:10, 0:16]
    h = _sp_bf(_dot(m, en1_ref) + b[1:2, 0:64])
    h = _sp_bf(_dot(h, en2_ref) + b[2:3, 0:64])
    en = _dot(h, en3_ref) + b[3:4, 0:2]

    zeros = jnp.zeros((en.shape[0], 128 - 22), jnp.float32)
    vals = jnp.concatenate([en, pp, pd, zeros], axis=1)
    part = lax.dot_general(onehotT, vals, (((1,), (0,)), ((), ())),
                           preferred_element_type=jnp.float32)
    out_ref[...] += part

    @pl.when(step == _GRID - 1)
    def _():
        acc = out_ref[...]
        cnt = jnp.maximum(acc[:, 1:2], 1.0)
        col = lax.broadcasted_iota(jnp.int32, (8, 128), 1)
        div = jnp.logical_and(col >= 2, col <= 21)
        out_ref[...] = jnp.where(div, acc / cnt, acc)


def _main(g, featT, src3, batch2d, f44, wp, bias, en1, en2, en3,
          p1, p2, p3, d1, d2, d3):
    full = lambda shape: pl.BlockSpec(shape, lambda i: (0,) * len(shape))
    return pl.pallas_call(
        _main_body,
        grid=(_GRID,),
        in_specs=[
            pl.BlockSpec((_BLK, 128), lambda i: (i, 0)),
            pl.BlockSpec((19, _BLK), lambda i: (0, i)),
            pl.BlockSpec((1, 1, _BLK), lambda i: (i, 0, 0)),
            full((80, 128)),
            full((8, 4)),
            full((24, 128)),
            full((16, 128)),
            full((128, 64)),
            full((64, 64)),
            full((64, 2)),
            full((128, 128)),
            full((128, 128)),
            full((128, 4)),
            full((128, 128)),
            full((128, 128)),
            full((128, 16)),
        ],
        out_specs=pl.BlockSpec((8, 128), lambda i: (0, 0)),
        out_shape=jax.ShapeDtypeStruct((8, 128), jnp.float32),
    )(g, featT, src3, batch2d, f44, wp, bias, en1, en2, en3,
      p1, p2, p3, d1, d2, d3)


def kernel(x, edge_attr, F, pos, r, d, mean_pos, params, edge_index, batch):
    p = params
    msg_W = p['msg_W']
    wj = msg_W[0:128]
    wi = msg_W[128:256]
    wp = jnp.zeros((24, 128), jnp.float32)
    wp = wp.at[0:22].set(msg_W[256:278])
    wp = wp.at[22].set(p['msg_b'])

    src = edge_index[0].astype(jnp.int32)
    dst = edge_index[1].astype(jnp.int32)
    featT = jnp.concatenate([r.T, d.T, edge_attr.T], axis=0)
    src3 = src.reshape(_GRID, 1, _BLK)
    idx4 = jnp.stack([src.reshape(_NW, _K, _C), dst.reshape(_NW, _K, _C)],
                     axis=2)

    batch2d = jnp.concatenate(
        [batch.astype(jnp.int32), jnp.full((_NPAD - N,), B, jnp.int32)]
    ).reshape(80, 128)

    f44 = jnp.zeros((8, 4), jnp.float32).at[0:4].set(F.reshape(4, 4))

    bias = jnp.zeros((16, 128), jnp.float32)
    bias = bias.at[0, :].set(p['msg_b'])
    bias = bias.at[1, 0:64].set(p['en1_b'])
    bias = bias.at[2, 0:64].set(p['en2_b'])
    bias = bias.at[3, 0:1].set(p['en3_b'])
    bias = bias.at[3, 1].set(1.0)
    bias = bias.at[4, :].set(p['P1_b'])
    bias = bias.at[5, :].set(p['P2_b'])
    bias = bias.at[6, 0:4].set(p['P3_b'])
    bias = bias.at[7, :].set(p['D1_b'])
    bias = bias.at[8, :].set(p['D2_b'])
    bias = bias.at[9, 0:16].set(p['D3_b'])

    en3p = jnp.zeros((64, 2), jnp.float32).at[:, 0:1].set(p['en3_W'])

    xa, xb = _project(x, wj, wi)
    g = _gather_add(xa, xb, idx4)
    bf = jnp.bfloat16
    res = _main(g, featT, src3, batch2d, f44, wp, bias,
                p['en1_W'].astype(bf), p['en2_W'].astype(bf), en3p.astype(bf),
                p['P1_W'].astype(bf), p['P2_W'].astype(bf),
                p['P3_W'].astype(bf),
                p['D1_W'].astype(bf), p['D2_W'].astype(bf),
                p['D3_W'].astype(bf))
    return jnp.concatenate([res[0:4, 0:1], res[0:4, 2:22]], axis=1)
